# baseline (speedup 1.0000x reference)
import functools

import jax
import jax.numpy as jnp
from jax.experimental import pallas as pl
from jax.experimental.pallas import tpu as pltpu
from jax.experimental.pallas import tpu_sc as plsc

_CHUNK = 64
_NBUF = 10
_LAG = 4


def kernel(x, table):
    B, S = x.shape
    V, D = table.shape
    n = B * S
    NW = 32
    per_w = n // NW
    nch = per_w // _CHUNK
    assert per_w % _CHUNK == 0 and nch % _NBUF == 0 and _LAG < _NBUF
    idx = x.reshape(NW, nch, _CHUNK).astype(jnp.int32)

    mesh = plsc.VectorSubcoreMesh(
        core_axis_name="core", subcore_axis_name="subcore"
    )

    scratch = (
        [pltpu.VMEM((nch, _CHUNK), jnp.int32)]
        + [pltpu.VMEM((_CHUNK, D), jnp.float32) for _ in range(_NBUF)]
        + [pltpu.SemaphoreType.DMA for _ in range(2 * _NBUF)]
    )

    @functools.partial(
        pl.kernel,
        out_type=jax.ShapeDtypeStruct((n, D), table.dtype),
        mesh=mesh,
        scratch_types=scratch,
    )
    def gather_kernel(tab_hbm, idx_hbm, out_hbm, idx_v, *rest):
        bufs = rest[:_NBUF]
        gsem = rest[_NBUF : 2 * _NBUF]
        wsem = rest[2 * _NBUF :]
        wid = jax.lax.axis_index("subcore") * 2 + jax.lax.axis_index("core")
        base = wid * per_w

        pltpu.sync_copy(idx_hbm.at[wid], idx_v)

        def start_gather(g, b):
            return pltpu.async_copy(
                tab_hbm.at[idx_v.at[g]], bufs[b], gsem[b]
            )

        def start_write(g, b):
            return pltpu.async_copy(
                bufs[b], out_hbm.at[pl.ds(base + g * _CHUNK, _CHUNK)], wsem[b]
            )

        def wait_gather(g, b):
            pltpu.make_async_copy(tab_hbm.at[idx_v.at[g]], bufs[b], gsem[b]).wait()

        def wait_write(g, b):
            pltpu.make_async_copy(
                bufs[b], out_hbm.at[pl.ds(base + g * _CHUNK, _CHUNK)], wsem[b]
            ).wait()

        for b in range(_NBUF):
            start_gather(b, b)
            if b >= _LAG:
                g2 = b - _LAG
                wait_gather(g2, g2 % _NBUF)
                start_write(g2, g2 % _NBUF)

        @pl.loop(1, nch // _NBUF)
        def _(grp):
            g0 = grp * _NBUF
            for b in range(_NBUF):
                g = g0 + b
                wait_write(g - _NBUF, b)
                start_gather(g, b)
                b2 = (b - _LAG) % _NBUF
                wait_gather(g - _LAG, b2)
                start_write(g - _LAG, b2)

        for k in range(_LAG):
            g2 = nch - _LAG + k
            b2 = g2 % _NBUF
            wait_gather(g2, b2)
            start_write(g2, b2)
        for k in range(_NBUF):
            g = nch - _NBUF + k
            wait_write(g, g % _NBUF)

    out = gather_kernel(table, idx)
    return out.reshape(B, S, D)


# speedup vs baseline: 1.0055x; 1.0055x over previous
import functools

import jax
import jax.numpy as jnp
from jax.experimental import pallas as pl
from jax.experimental.pallas import tpu as pltpu
from jax.experimental.pallas import tpu_sc as plsc

_CHUNK = 128
_NBUF = 5
_LAG = 3


def kernel(x, table):
    B, S = x.shape
    V, D = table.shape
    n = B * S
    NW = 32
    per_w = n // NW
    nch = per_w // _CHUNK
    assert per_w % _CHUNK == 0 and nch % _NBUF == 0 and _LAG < _NBUF
    idx = x.reshape(NW, nch, _CHUNK).astype(jnp.int32)

    mesh = plsc.VectorSubcoreMesh(
        core_axis_name="core", subcore_axis_name="subcore"
    )

    scratch = (
        [pltpu.VMEM((nch, _CHUNK), jnp.int32)]
        + [pltpu.VMEM((_CHUNK, D), jnp.float32) for _ in range(_NBUF)]
        + [pltpu.SemaphoreType.DMA for _ in range(2 * _NBUF + 1)]
    )

    @functools.partial(
        pl.kernel,
        out_type=jax.ShapeDtypeStruct((n, D), table.dtype),
        mesh=mesh,
        scratch_types=scratch,
    )
    def gather_kernel(tab_hbm, idx_hbm, out_hbm, idx_v, *rest):
        bufs = rest[:_NBUF]
        gsem = rest[_NBUF : 2 * _NBUF]
        wsem = rest[2 * _NBUF : 3 * _NBUF]
        isem = rest[3 * _NBUF]
        wid = jax.lax.axis_index("subcore") * 2 + jax.lax.axis_index("core")
        base = wid * per_w

        head = 8
        pltpu.sync_copy(
            idx_hbm.at[wid, pl.ds(0, head)], idx_v.at[pl.ds(0, head)]
        )
        rest_idx = pltpu.async_copy(
            idx_hbm.at[wid, pl.ds(head, nch - head)],
            idx_v.at[pl.ds(head, nch - head)],
            isem,
        )

        def start_gather(g, b):
            return pltpu.async_copy(
                tab_hbm.at[idx_v.at[g]], bufs[b], gsem[b]
            )

        def start_write(g, b):
            return pltpu.async_copy(
                bufs[b], out_hbm.at[pl.ds(base + g * _CHUNK, _CHUNK)], wsem[b]
            )

        def wait_gather(g, b):
            pltpu.make_async_copy(tab_hbm.at[idx_v.at[g]], bufs[b], gsem[b]).wait()

        def wait_write(g, b):
            pltpu.make_async_copy(
                bufs[b], out_hbm.at[pl.ds(base + g * _CHUNK, _CHUNK)], wsem[b]
            ).wait()

        for b in range(_NBUF):
            start_gather(b, b)
            if b >= _LAG:
                g2 = b - _LAG
                wait_gather(g2, g2 % _NBUF)
                start_write(g2, g2 % _NBUF)

        rest_idx.wait()

        @pl.loop(1, nch // _NBUF)
        def _(grp):
            g0 = grp * _NBUF
            for b in range(_NBUF):
                g = g0 + b
                wait_write(g - _NBUF, b)
                start_gather(g, b)
                b2 = (b - _LAG) % _NBUF
                wait_gather(g - _LAG, b2)
                start_write(g - _LAG, b2)

        for k in range(_LAG):
            g2 = nch - _LAG + k
            b2 = g2 % _NBUF
            wait_gather(g2, b2)
            start_write(g2, b2)
        for k in range(_NBUF):
            g = nch - _NBUF + k
            wait_write(g, g % _NBUF)

    out = gather_kernel(table, idx)
    return out.reshape(B, S, D)
